# baseline (speedup 1.0000x reference)
import functools

import jax
import jax.numpy as jnp
from jax import lax
from jax.experimental import pallas as pl
from jax.experimental.pallas import tpu as pltpu
from jax.experimental.pallas import tpu_sc as plsc

B = 262144
W = 256
NOP = 64
L = 16

_info = plsc.get_sparse_core_info()
NC = _info.num_cores
NS = _info.num_subcores
NW = NC * NS
RPW = B // NW
CH = 64
NCH = RPW // CH
NBUF = 4
PF = 2
NJ = NCH // NBUF

_mesh = plsc.VectorSubcoreMesh(core_axis_name="c", subcore_axis_name="s")


@functools.partial(
    pl.kernel,
    mesh=_mesh,
    out_type=jax.ShapeDtypeStruct((B, W), jnp.float32),
    scratch_types=[
        pltpu.VMEM((NOP, W), jnp.float32),
        pltpu.VMEM((RPW,), jnp.int32),
        pltpu.VMEM((RPW,), jnp.float32),
        pltpu.VMEM((NBUF, CH, W), jnp.float32),
    ] + [pltpu.SemaphoreType.DMA] * (2 * NBUF),
)
def _sc_write_action(state_hbm, maskf_hbm, op_hbm, pred_hbm, out_hbm,
                     mask_v, op_v, pred_v, buf_v, *sems):
    wid = lax.axis_index("s") * NC + lax.axis_index("c")
    base = wid * RPW
    in_sems = sems[:NBUF]
    out_sems = sems[NBUF:]

    pltpu.sync_copy(maskf_hbm, mask_v)
    pltpu.sync_copy(op_hbm.at[pl.ds(base, RPW)], op_v)
    pltpu.sync_copy(pred_hbm.at[pl.ds(base, RPW)], pred_v)

    def in_dma(k, b):
        return pltpu.make_async_copy(
            state_hbm.at[pl.ds(base + k * CH, CH)], buf_v.at[b], in_sems[b])

    def out_dma(k, b):
        return pltpu.make_async_copy(
            buf_v.at[b], out_hbm.at[pl.ds(base + k * CH, CH)], out_sems[b])

    def compute_chunk(k, b):
        @plsc.parallel_loop(0, CH // L, unroll=1)
        def group_body(g):
            opvec = op_v[pl.ds(k * CH + g * L, L)]
            prvec = pred_v[pl.ds(k * CH + g * L, L)]
            for rr in range(L):
                r = g * L + rr
                op = opvec[rr]
                pv = jnp.full((L,), prvec[rr], jnp.float32)
                res = []
                for c in range(W // L):
                    m = mask_v[op, pl.ds(c * L, L)]
                    s = buf_v[b, r, pl.ds(c * L, L)]
                    res.append(jnp.where(m > 0.5, pv, s))
                for c in range(W // L):
                    buf_v[b, r, pl.ds(c * L, L)] = res[c]

    for k0 in range(PF):
        in_dma(k0, k0).start()

    def ring_body(j, carry):
        for b in range(NBUF):
            k = j * NBUF + b

            b2 = (b + PF) % NBUF

            @pl.when(k + PF < NCH)
            def _prefetch():
                @pl.when(k + PF >= NBUF)
                def _wait_prev_out():
                    out_dma(k + PF - NBUF, b2).wait()
                in_dma(k + PF, b2).start()

            in_dma(k, b).wait()
            compute_chunk(k, b)
            out_dma(k, b).start()
        return carry

    lax.fori_loop(0, NJ, ring_body, 0)

    for k in range(NCH - NBUF, NCH):
        out_dma(k, k % NBUF).wait()


def kernel(state_tensor, write_mask, operation, prediction):
    maskf = write_mask.astype(jnp.float32)
    opi = operation.astype(jnp.int32)
    predf = prediction.astype(jnp.float32)
    return _sc_write_action(state_tensor, maskf, opi, predf)
